# baseline (speedup 1.0000x reference)
import functools

import jax
import jax.numpy as jnp
from jax import lax
from jax.experimental import pallas as pl
from jax.experimental.pallas import tpu as pltpu
from jax.experimental.pallas import tpu_sc as plsc

N_NODES = 10000
N_EDGES = 320000
D = 128
D5 = 5 * D
ATTR = 33

EDGE_BLOCK = 1280

NC = 2
NS = 16
NW = NC * NS
EPW = N_EDGES // NW
CH = 80
NCHUNK = EPW // CH
NP = 10240

_SC_MESH = dict(core_axis_name="c", subcore_axis_name="s")


def _gather_body(xn_hbm, src_hbm, dst_hbm, xs_hbm, xd_hbm,
                 idx_v, rows_v, sem):
    wid = lax.axis_index("s") * NC + lax.axis_index("c")

    def body(i, carry):
        off = wid * EPW + i * CH
        pltpu.sync_copy(src_hbm.at[pl.ds(off, CH)], idx_v)
        pltpu.async_copy(xn_hbm.at[idx_v], rows_v, sem).wait()
        pltpu.sync_copy(rows_v, xs_hbm.at[pl.ds(off, CH)])
        pltpu.sync_copy(dst_hbm.at[pl.ds(off, CH)], idx_v)
        pltpu.async_copy(xn_hbm.at[idx_v], rows_v, sem).wait()
        pltpu.sync_copy(rows_v, xd_hbm.at[pl.ds(off, CH)])
        return carry

    lax.fori_loop(0, NCHUNK, body, 0)


def _sc_gather(xn, xe_src, xe_dst):
    return pl.kernel(
        _gather_body,
        out_type=[
            jax.ShapeDtypeStruct((N_EDGES, D), jnp.float32),
            jax.ShapeDtypeStruct((N_EDGES, D), jnp.float32),
        ],
        mesh=plsc.VectorSubcoreMesh(**_SC_MESH),
        scratch_types=[
            pltpu.VMEM((CH,), jnp.int32),
            pltpu.VMEM((CH, D), jnp.float32),
            pltpu.SemaphoreType.DMA,
        ],
    )(xn, xe_src, xe_dst)


def _scatter_body(md_hbm, ms_hbm, dst_hbm, src_hbm, zeros_hbm, out_hbm,
                  idx_v, rows_v, acc_sh):
    cid = lax.axis_index("c")
    sid = lax.axis_index("s")
    wid = sid * NC + cid
    zr = NP // NS
    pltpu.sync_copy(zeros_hbm.at[pl.ds(sid * zr, zr)],
                    acc_sh.at[pl.ds(sid * zr, zr)])
    plsc.subcore_barrier()

    def body(i, carry):
        off = wid * EPW + i * CH
        pltpu.sync_copy(dst_hbm.at[pl.ds(off, CH)], idx_v)
        pltpu.sync_copy(md_hbm.at[pl.ds(off, CH)], rows_v)
        pltpu.sync_copy(rows_v, acc_sh.at[idx_v], add=True)
        pltpu.sync_copy(src_hbm.at[pl.ds(off, CH)], idx_v)
        pltpu.sync_copy(ms_hbm.at[pl.ds(off, CH)], rows_v)
        pltpu.sync_copy(rows_v, acc_sh.at[idx_v], add=True)
        return carry

    lax.fori_loop(0, NCHUNK, body, 0)
    plsc.subcore_barrier()
    pltpu.sync_copy(acc_sh.at[pl.ds(sid * zr, zr)],
                    out_hbm.at[cid].at[pl.ds(sid * zr, zr)])


def _sc_scatter(md, ms, xe_dst, xe_src):
    zeros = jnp.zeros((NP, D), jnp.float32)
    return pl.kernel(
        _scatter_body,
        out_type=jax.ShapeDtypeStruct((NC, NP, D), jnp.float32),
        mesh=plsc.VectorSubcoreMesh(**_SC_MESH),
        scratch_types=[
            pltpu.VMEM((CH,), jnp.int32),
            pltpu.VMEM((CH, D), jnp.float32),
            pltpu.VMEM_SHARED((NP, D), jnp.float32),
        ],
    )(md, ms, xe_dst, xe_src, zeros)


def _edge_block_kernel(attr_ref, xs_ref, xd_ref, w1p_ref, b_ref, wt_ref,
                       md_ref, ms_ref):
    attr = attr_ref[...]
    w = jax.nn.silu(
        jnp.dot(attr, w1p_ref[...], preferred_element_type=jnp.float32)
        + b_ref[...])
    xs = xs_ref[...]
    xd = xd_ref[...]
    g = w * (xs - xd)
    a = 0.5 * w * (xs + xd)
    dxe = jnp.concatenate([g, a, g * a, g * g, a * a], axis=1)
    wt = wt_ref[...]
    x = jnp.tanh(dxe).astype(jnp.bfloat16)
    x = jnp.dot(x, wt, preferred_element_type=jnp.float32)
    x = x - jnp.mean(x, axis=1, keepdims=True)
    x = x * lax.rsqrt(jnp.sum(x * x, axis=1, keepdims=True) + 0.001)
    x = jnp.tanh(x).astype(jnp.bfloat16)
    x = jnp.dot(x, wt, preferred_element_type=jnp.float32)
    dxe2 = jnp.tanh(x)
    g2 = w * dxe2[:, :D]
    s2 = 0.5 * w * (dxe2[:, D:2 * D] + dxe2[:, 2 * D:3 * D]
                    + dxe2[:, 3 * D:4 * D] + dxe2[:, 4 * D:])
    md_ref[...] = g2 + s2
    ms_ref[...] = s2 - g2


def _edge_mlp(xe_attr, xs, xd, fc1_w, fc1_b, dl_w1, *, interpret=False):
    nb = N_EDGES // EDGE_BLOCK
    w1p = fc1_w.T
    b = fc1_b.reshape(1, D)
    wt = dl_w1.T.astype(jnp.bfloat16)
    md, ms = pl.pallas_call(
        _edge_block_kernel,
        grid=(nb,),
        in_specs=[
            pl.BlockSpec((EDGE_BLOCK, ATTR), lambda i: (i, 0)),
            pl.BlockSpec((EDGE_BLOCK, D), lambda i: (i, 0)),
            pl.BlockSpec((EDGE_BLOCK, D), lambda i: (i, 0)),
            pl.BlockSpec((ATTR, D), lambda i: (0, 0)),
            pl.BlockSpec((1, D), lambda i: (0, 0)),
            pl.BlockSpec((D5, D5), lambda i: (0, 0)),
        ],
        out_specs=[
            pl.BlockSpec((EDGE_BLOCK, D), lambda i: (i, 0)),
            pl.BlockSpec((EDGE_BLOCK, D), lambda i: (i, 0)),
        ],
        out_shape=[
            jax.ShapeDtypeStruct((N_EDGES, D), jnp.float32),
            jax.ShapeDtypeStruct((N_EDGES, D), jnp.float32),
        ],
        interpret=interpret,
    )(xe_attr, xs, xd, w1p, b, wt)
    return md, ms


def kernel(xn, xe_attr, xe_src, xe_dst, fc1_w, fc1_b, dl_w1, dl_w2):
    del dl_w2
    xs, xd = _sc_gather(xn, xe_src, xe_dst)
    md, ms = _edge_mlp(xe_attr, xs, xd, fc1_w, fc1_b, dl_w1)
    partials = _sc_scatter(md, ms, xe_dst, xe_src)
    return (partials[0] + partials[1])[:N_NODES]


# speedup vs baseline: 1.4417x; 1.4417x over previous
import functools

import jax
import jax.numpy as jnp
from jax import lax
from jax.experimental import pallas as pl
from jax.experimental.pallas import tpu as pltpu
from jax.experimental.pallas import tpu_sc as plsc

N_NODES = 10000
N_EDGES = 320000
D = 128
D5 = 5 * D
ATTR = 33

EDGE_BLOCK = 1280

NC = 2
NS = 16
NW = NC * NS
CH = 80
NP = 10240

N_SLAB = 5
E_SLAB = N_EDGES // N_SLAB
EPW = E_SLAB // NW
NCHUNK = EPW // CH

_SC_MESH = dict(core_axis_name="c", subcore_axis_name="s")


def _gather_body(xn_hbm, src_hbm, dst_hbm, xs_hbm, xd_hbm,
                 idx_v, rows_v, sem):
    wid = lax.axis_index("s") * NC + lax.axis_index("c")

    def body(i, carry):
        off = wid * EPW + i * CH
        pltpu.sync_copy(src_hbm.at[pl.ds(off, CH)], idx_v)
        pltpu.async_copy(xn_hbm.at[idx_v], rows_v, sem).wait()
        pltpu.sync_copy(rows_v, xs_hbm.at[pl.ds(off, CH)])
        pltpu.sync_copy(dst_hbm.at[pl.ds(off, CH)], idx_v)
        pltpu.async_copy(xn_hbm.at[idx_v], rows_v, sem).wait()
        pltpu.sync_copy(rows_v, xd_hbm.at[pl.ds(off, CH)])
        return carry

    lax.fori_loop(0, NCHUNK, body, 0)


def _sc_gather(xn, xe_src, xe_dst):
    return pl.kernel(
        _gather_body,
        out_type=[
            jax.ShapeDtypeStruct((E_SLAB, D), jnp.float32),
            jax.ShapeDtypeStruct((E_SLAB, D), jnp.float32),
        ],
        mesh=plsc.VectorSubcoreMesh(**_SC_MESH),
        scratch_types=[
            pltpu.VMEM((CH,), jnp.int32),
            pltpu.VMEM((CH, D), jnp.float32),
            pltpu.SemaphoreType.DMA,
        ],
    )(xn, xe_src, xe_dst)


def _scatter_body(md_hbm, ms_hbm, dst_hbm, src_hbm, zeros_hbm, out_hbm,
                  idx_v, rows_v, acc_sh):
    cid = lax.axis_index("c")
    sid = lax.axis_index("s")
    wid = sid * NC + cid
    zr = NP // NS
    pltpu.sync_copy(zeros_hbm.at[pl.ds(sid * zr, zr)],
                    acc_sh.at[pl.ds(sid * zr, zr)])
    plsc.subcore_barrier()

    def body(i, carry):
        off = wid * EPW + i * CH
        pltpu.sync_copy(dst_hbm.at[pl.ds(off, CH)], idx_v)
        pltpu.sync_copy(md_hbm.at[pl.ds(off, CH)], rows_v)
        pltpu.sync_copy(rows_v, acc_sh.at[idx_v], add=True)
        pltpu.sync_copy(src_hbm.at[pl.ds(off, CH)], idx_v)
        pltpu.sync_copy(ms_hbm.at[pl.ds(off, CH)], rows_v)
        pltpu.sync_copy(rows_v, acc_sh.at[idx_v], add=True)
        return carry

    lax.fori_loop(0, NCHUNK, body, 0)
    plsc.subcore_barrier()
    pltpu.sync_copy(acc_sh.at[pl.ds(sid * zr, zr)],
                    out_hbm.at[cid].at[pl.ds(sid * zr, zr)])


def _sc_scatter(md, ms, xe_dst, xe_src, zeros):
    return pl.kernel(
        _scatter_body,
        out_type=jax.ShapeDtypeStruct((NC, NP, D), jnp.float32),
        mesh=plsc.VectorSubcoreMesh(**_SC_MESH),
        scratch_types=[
            pltpu.VMEM((CH,), jnp.int32),
            pltpu.VMEM((CH, D), jnp.float32),
            pltpu.VMEM_SHARED((NP, D), jnp.float32),
        ],
    )(md, ms, xe_dst, xe_src, zeros)


def _edge_block_kernel(attr_ref, xs_ref, xd_ref, w1p_ref, b_ref, wt_ref,
                       md_ref, ms_ref):
    attr = attr_ref[...]
    w = jax.nn.silu(
        jnp.dot(attr, w1p_ref[...], preferred_element_type=jnp.float32)
        + b_ref[...])
    xs = xs_ref[...]
    xd = xd_ref[...]
    g = w * (xs - xd)
    a = 0.5 * w * (xs + xd)
    dxe = jnp.concatenate([g, a, g * a, g * g, a * a], axis=1)
    wt = wt_ref[...]
    x = jnp.tanh(dxe).astype(jnp.bfloat16)
    x = jnp.dot(x, wt, preferred_element_type=jnp.float32)
    x = x - jnp.mean(x, axis=1, keepdims=True)
    x = x * lax.rsqrt(jnp.sum(x * x, axis=1, keepdims=True) + 0.001)
    x = jnp.tanh(x).astype(jnp.bfloat16)
    x = jnp.dot(x, wt, preferred_element_type=jnp.float32)
    dxe2 = jnp.tanh(x)
    g2 = w * dxe2[:, :D]
    s2 = 0.5 * w * (dxe2[:, D:2 * D] + dxe2[:, 2 * D:3 * D]
                    + dxe2[:, 3 * D:4 * D] + dxe2[:, 4 * D:])
    md_ref[...] = g2 + s2
    ms_ref[...] = s2 - g2


def _edge_mlp(xe_attr, xs, xd, w1p, b, wt, *, interpret=False):
    nb = xe_attr.shape[0] // EDGE_BLOCK
    md, ms = pl.pallas_call(
        _edge_block_kernel,
        grid=(nb,),
        in_specs=[
            pl.BlockSpec((EDGE_BLOCK, ATTR), lambda i: (i, 0)),
            pl.BlockSpec((EDGE_BLOCK, D), lambda i: (i, 0)),
            pl.BlockSpec((EDGE_BLOCK, D), lambda i: (i, 0)),
            pl.BlockSpec((ATTR, D), lambda i: (0, 0)),
            pl.BlockSpec((1, D), lambda i: (0, 0)),
            pl.BlockSpec((D5, D5), lambda i: (0, 0)),
        ],
        out_specs=[
            pl.BlockSpec((EDGE_BLOCK, D), lambda i: (i, 0)),
            pl.BlockSpec((EDGE_BLOCK, D), lambda i: (i, 0)),
        ],
        out_shape=[
            jax.ShapeDtypeStruct((xe_attr.shape[0], D), jnp.float32),
            jax.ShapeDtypeStruct((xe_attr.shape[0], D), jnp.float32),
        ],
        interpret=interpret,
    )(xe_attr, xs, xd, w1p, b, wt)
    return md, ms


def kernel(xn, xe_attr, xe_src, xe_dst, fc1_w, fc1_b, dl_w1, dl_w2):
    del dl_w2
    w1p = fc1_w.T
    b = fc1_b.reshape(1, D)
    wt = dl_w1.T.astype(jnp.bfloat16)
    zeros = jnp.zeros((NP, D), jnp.float32)
    partials = []
    for s in range(N_SLAB):
        sl = slice(s * E_SLAB, (s + 1) * E_SLAB)
        src_s, dst_s = xe_src[sl], xe_dst[sl]
        xs, xd = _sc_gather(xn, src_s, dst_s)
        md, ms = _edge_mlp(xe_attr[sl], xs, xd, w1p, b, wt)
        partials.append(_sc_scatter(md, ms, dst_s, src_s, zeros))
    acc = sum(p[0] + p[1] for p in partials)
    return acc[:N_NODES]


# speedup vs baseline: 1.4825x; 1.0283x over previous
import functools

import jax
import jax.numpy as jnp
from jax import lax
from jax.experimental import pallas as pl
from jax.experimental.pallas import tpu as pltpu
from jax.experimental.pallas import tpu_sc as plsc

N_NODES = 10000
N_EDGES = 320000
D = 128
D5 = 5 * D
ATTR = 33

EDGE_BLOCK = 1280

NC = 2
NS = 16
NW = NC * NS
CH = 80
NP = 10240

N_SLAB = 5
E_SLAB = N_EDGES // N_SLAB
EPW = E_SLAB // NW
NCHUNK = EPW // CH

_SC_MESH = dict(core_axis_name="c", subcore_axis_name="s")


def _gather_body(xn_hbm, src_hbm, dst_hbm, xs_hbm, xd_hbm,
                 idx_v, rows_v, xn_sh, sem):
    cid = lax.axis_index("c")
    sid = lax.axis_index("s")
    wid = sid * NC + cid
    @pl.when(sid < NS - 1)
    def _():
        pltpu.sync_copy(xn_hbm.at[pl.ds(sid * 640, 640)],
                        xn_sh.at[pl.ds(sid * 640, 640)])

    @pl.when(sid == NS - 1)
    def _():
        pltpu.sync_copy(xn_hbm.at[pl.ds((NS - 1) * 640, N_NODES - (NS - 1) * 640)],
                        xn_sh.at[pl.ds((NS - 1) * 640, N_NODES - (NS - 1) * 640)])

    plsc.subcore_barrier()

    def body(i, carry):
        off = wid * EPW + i * CH
        pltpu.sync_copy(src_hbm.at[pl.ds(off, CH)], idx_v)
        pltpu.sync_copy(xn_sh.at[idx_v], rows_v)
        pltpu.sync_copy(rows_v, xs_hbm.at[pl.ds(off, CH)])
        pltpu.sync_copy(dst_hbm.at[pl.ds(off, CH)], idx_v)
        pltpu.sync_copy(xn_sh.at[idx_v], rows_v)
        pltpu.sync_copy(rows_v, xd_hbm.at[pl.ds(off, CH)])
        return carry

    lax.fori_loop(0, NCHUNK, body, 0)


def _sc_gather(xn, xe_src, xe_dst):
    return pl.kernel(
        _gather_body,
        out_type=[
            jax.ShapeDtypeStruct((E_SLAB, D), jnp.float32),
            jax.ShapeDtypeStruct((E_SLAB, D), jnp.float32),
        ],
        mesh=plsc.VectorSubcoreMesh(**_SC_MESH),
        scratch_types=[
            pltpu.VMEM((CH,), jnp.int32),
            pltpu.VMEM((CH, D), jnp.float32),
            pltpu.VMEM_SHARED((N_NODES, D), jnp.float32),
            pltpu.SemaphoreType.DMA,
        ],
    )(xn, xe_src, xe_dst)


def _scatter_body(md_hbm, ms_hbm, dst_hbm, src_hbm, zeros_hbm, out_hbm,
                  idx_v, rows_v, acc_sh):
    cid = lax.axis_index("c")
    sid = lax.axis_index("s")
    wid = sid * NC + cid
    zr = NP // NS
    pltpu.sync_copy(zeros_hbm.at[pl.ds(sid * zr, zr)],
                    acc_sh.at[pl.ds(sid * zr, zr)])
    plsc.subcore_barrier()

    def body(i, carry):
        off = wid * EPW + i * CH
        pltpu.sync_copy(dst_hbm.at[pl.ds(off, CH)], idx_v)
        pltpu.sync_copy(md_hbm.at[pl.ds(off, CH)], rows_v)
        pltpu.sync_copy(rows_v, acc_sh.at[idx_v], add=True)
        pltpu.sync_copy(src_hbm.at[pl.ds(off, CH)], idx_v)
        pltpu.sync_copy(ms_hbm.at[pl.ds(off, CH)], rows_v)
        pltpu.sync_copy(rows_v, acc_sh.at[idx_v], add=True)
        return carry

    lax.fori_loop(0, NCHUNK, body, 0)
    plsc.subcore_barrier()
    pltpu.sync_copy(acc_sh.at[pl.ds(sid * zr, zr)],
                    out_hbm.at[cid].at[pl.ds(sid * zr, zr)])


def _sc_scatter(md, ms, xe_dst, xe_src, zeros):
    return pl.kernel(
        _scatter_body,
        out_type=jax.ShapeDtypeStruct((NC, NP, D), jnp.float32),
        mesh=plsc.VectorSubcoreMesh(**_SC_MESH),
        scratch_types=[
            pltpu.VMEM((CH,), jnp.int32),
            pltpu.VMEM((CH, D), jnp.float32),
            pltpu.VMEM_SHARED((NP, D), jnp.float32),
        ],
    )(md, ms, xe_dst, xe_src, zeros)


def _edge_block_kernel(attr_ref, xs_ref, xd_ref, w1p_ref, b_ref, wt_ref,
                       md_ref, ms_ref):
    attr = attr_ref[...]
    w = jax.nn.silu(
        jnp.dot(attr, w1p_ref[...], preferred_element_type=jnp.float32)
        + b_ref[...])
    xs = xs_ref[...]
    xd = xd_ref[...]
    g = w * (xs - xd)
    a = 0.5 * w * (xs + xd)
    dxe = jnp.concatenate([g, a, g * a, g * g, a * a], axis=1)
    wt = wt_ref[...]
    x = jnp.tanh(dxe).astype(jnp.bfloat16)
    x = jnp.dot(x, wt, preferred_element_type=jnp.float32)
    x = x - jnp.mean(x, axis=1, keepdims=True)
    x = x * lax.rsqrt(jnp.sum(x * x, axis=1, keepdims=True) + 0.001)
    x = jnp.tanh(x).astype(jnp.bfloat16)
    x = jnp.dot(x, wt, preferred_element_type=jnp.float32)
    dxe2 = jnp.tanh(x)
    g2 = w * dxe2[:, :D]
    s2 = 0.5 * w * (dxe2[:, D:2 * D] + dxe2[:, 2 * D:3 * D]
                    + dxe2[:, 3 * D:4 * D] + dxe2[:, 4 * D:])
    md_ref[...] = g2 + s2
    ms_ref[...] = s2 - g2


def _edge_mlp(xe_attr, xs, xd, w1p, b, wt, *, interpret=False):
    nb = xe_attr.shape[0] // EDGE_BLOCK
    md, ms = pl.pallas_call(
        _edge_block_kernel,
        grid=(nb,),
        in_specs=[
            pl.BlockSpec((EDGE_BLOCK, ATTR), lambda i: (i, 0)),
            pl.BlockSpec((EDGE_BLOCK, D), lambda i: (i, 0)),
            pl.BlockSpec((EDGE_BLOCK, D), lambda i: (i, 0)),
            pl.BlockSpec((ATTR, D), lambda i: (0, 0)),
            pl.BlockSpec((1, D), lambda i: (0, 0)),
            pl.BlockSpec((D5, D5), lambda i: (0, 0)),
        ],
        out_specs=[
            pl.BlockSpec((EDGE_BLOCK, D), lambda i: (i, 0)),
            pl.BlockSpec((EDGE_BLOCK, D), lambda i: (i, 0)),
        ],
        out_shape=[
            jax.ShapeDtypeStruct((xe_attr.shape[0], D), jnp.float32),
            jax.ShapeDtypeStruct((xe_attr.shape[0], D), jnp.float32),
        ],
        interpret=interpret,
    )(xe_attr, xs, xd, w1p, b, wt)
    return md, ms


def kernel(xn, xe_attr, xe_src, xe_dst, fc1_w, fc1_b, dl_w1, dl_w2):
    del dl_w2
    w1p = fc1_w.T
    b = fc1_b.reshape(1, D)
    wt = dl_w1.T.astype(jnp.bfloat16)

    zeros = jnp.zeros((NP, D), jnp.float32)
    partials = []
    for s in range(N_SLAB):
        sl = slice(s * E_SLAB, (s + 1) * E_SLAB)
        src_s, dst_s = xe_src[sl], xe_dst[sl]
        xs, xd = _sc_gather(xn, src_s, dst_s)
        md, ms = _edge_mlp(xe_attr[sl], xs, xd, w1p, b, wt)
        partials.append(_sc_scatter(md, ms, dst_s, src_s, zeros))
    acc = sum(p[0] + p[1] for p in partials)
    return acc[:N_NODES]


# speedup vs baseline: 1.5060x; 1.0159x over previous
import functools

import jax
import jax.numpy as jnp
from jax import lax
from jax.experimental import pallas as pl
from jax.experimental.pallas import tpu as pltpu
from jax.experimental.pallas import tpu_sc as plsc

N_NODES = 10000
N_EDGES = 320000
D = 128
D5 = 5 * D
ATTR = 33

EDGE_BLOCK = 1280

NC = 2
NS = 16
NW = NC * NS
CH = 80
NP = 10240

N_SLAB = 5
E_SLAB = N_EDGES // N_SLAB
EPW = E_SLAB // NW
NCHUNK = EPW // CH
GPR = 5
RB = GPR * CH
NGRP = EPW // RB
RB_S = CH
NGRP_S = EPW // RB_S

_SC_MESH = dict(core_axis_name="c", subcore_axis_name="s")


def _gather_body(xn_hbm, src_hbm, dst_hbm, xs_hbm, xd_hbm,
                 idxs_v, idxd_v, rows0_v, rows1_v,
                 gsem0, gsem1, wsem0, wsem1):
    cid = lax.axis_index("c")
    sid = lax.axis_index("s")
    wid = sid * NC + cid
    base = wid * EPW
    pltpu.sync_copy(src_hbm.at[wid], idxs_v)
    pltpu.sync_copy(dst_hbm.at[wid], idxd_v)

    rows = (rows0_v, rows1_v)
    gsems = (gsem0, gsem1)
    wsems = (wsem0, wsem1)
    nstep = 2 * NGRP
    plan = [(idxs_v, xs_hbm, t) if t < NGRP else (idxd_v, xd_hbm, t - NGRP)
            for t in range(nstep)]

    def fire_gathers(t):
        idxv, _, k = plan[t]
        buf, sem = rows[t % 2], gsems[t % 2]
        return [pltpu.async_copy(xn_hbm.at[idxv.at[GPR * k + b]],
                                 buf.at[pl.ds(b * CH, CH)], sem)
                for b in range(GPR)]

    wb = [None, None]
    pend = fire_gathers(0)
    for t in range(nstep):
        bt = t % 2
        for c in pend:
            c.wait()
        if t + 1 < nstep:
            if wb[1 - bt] is not None:
                wb[1 - bt].wait()
                wb[1 - bt] = None
            pend = fire_gathers(t + 1)
        _, out_hbm, k = plan[t]
        wb[bt] = pltpu.async_copy(
            rows[bt], out_hbm.at[pl.ds(base + k * RB, RB)], wsems[bt])
    for c in wb:
        if c is not None:
            c.wait()


def _sc_gather(xn, xe_src, xe_dst):
    return pl.kernel(
        _gather_body,
        out_type=[
            jax.ShapeDtypeStruct((E_SLAB, D), jnp.float32),
            jax.ShapeDtypeStruct((E_SLAB, D), jnp.float32),
        ],
        mesh=plsc.VectorSubcoreMesh(**_SC_MESH),
        scratch_types=[
            pltpu.VMEM((NCHUNK, CH), jnp.int32),
            pltpu.VMEM((NCHUNK, CH), jnp.int32),
            pltpu.VMEM((RB, D), jnp.float32),
            pltpu.VMEM((RB, D), jnp.float32),
            pltpu.SemaphoreType.DMA,
            pltpu.SemaphoreType.DMA,
            pltpu.SemaphoreType.DMA,
            pltpu.SemaphoreType.DMA,
        ],
    )(xn, xe_src, xe_dst)


def _scatter_body(md_hbm, ms_hbm, dst_hbm, src_hbm, tok_hbm, out_hbm,
                  idxd_v, idxs_v, rows0_v, rows1_v, acc_sh,
                  lsem0, lsem1, asem0, asem1):
    cid = lax.axis_index("c")
    sid = lax.axis_index("s")
    wid = sid * NC + cid
    base = wid * EPW
    zr = NP // NS
    pltpu.sync_copy(dst_hbm.at[wid], idxd_v)
    pltpu.sync_copy(src_hbm.at[wid], idxs_v)
    zvec = jnp.zeros((16,), jnp.float32)
    for r in range(16):
        for c in range(D // 16):
            rows0_v[r, pl.ds(c * 16, 16)] = zvec
    for z in range(zr // 16):
        pltpu.sync_copy(rows0_v.at[pl.ds(0, 16)],
                        acc_sh.at[pl.ds(sid * zr + z * 16, 16)])
    plsc.subcore_barrier()

    rows = (rows0_v, rows1_v)
    lsems = (lsem0, lsem1)
    asems = (asem0, asem1)
    nstep = 2 * NGRP_S
    plan = [(idxd_v, md_hbm, t) if t < NGRP_S else (idxs_v, ms_hbm, t - NGRP_S)
            for t in range(nstep)]

    def fire_load(t):
        _, m_hbm, k = plan[t]
        return pltpu.async_copy(m_hbm.at[pl.ds(base + k * RB_S, RB_S)],
                                rows[t % 2], lsems[t % 2])

    pend = fire_load(0)
    for t in range(nstep):
        bt = t % 2
        nxt = fire_load(t + 1) if t + 1 < nstep else None
        pend.wait()
        pend = nxt
        idxv, _, k = plan[t]
        pltpu.sync_copy(rows[bt], acc_sh.at[idxv.at[k]], add=True)
    plsc.subcore_barrier()
    pltpu.sync_copy(acc_sh.at[pl.ds(sid * zr, zr)],
                    out_hbm.at[cid].at[pl.ds(sid * zr, zr)])


def _sc_scatter(md, ms, xe_dst, xe_src, tok):
    return pl.kernel(
        _scatter_body,
        out_type=jax.ShapeDtypeStruct((NC, NP, D), jnp.float32),
        mesh=plsc.VectorSubcoreMesh(**_SC_MESH),
        scratch_types=[
            pltpu.VMEM((NCHUNK, CH), jnp.int32),
            pltpu.VMEM((NCHUNK, CH), jnp.int32),
            pltpu.VMEM((RB_S, D), jnp.float32),
            pltpu.VMEM((RB_S, D), jnp.float32),
            pltpu.VMEM_SHARED((NP, D), jnp.float32),
            pltpu.SemaphoreType.DMA,
            pltpu.SemaphoreType.DMA,
            pltpu.SemaphoreType.DMA,
            pltpu.SemaphoreType.DMA,
        ],
    )(md, ms, xe_dst, xe_src, tok)


def _edge_block_kernel(attr_ref, xs_ref, xd_ref, w1p_ref, b_ref, wt_ref,
                       md_ref, ms_ref):
    attr = attr_ref[...]
    w = jax.nn.silu(
        jnp.dot(attr, w1p_ref[...], preferred_element_type=jnp.float32)
        + b_ref[...])
    xs = xs_ref[...]
    xd = xd_ref[...]
    g = w * (xs - xd)
    a = 0.5 * w * (xs + xd)
    dxe = jnp.concatenate([g, a, g * a, g * g, a * a], axis=1)
    wt = wt_ref[...]
    x = jnp.tanh(dxe).astype(jnp.bfloat16)
    x = jnp.dot(x, wt, preferred_element_type=jnp.float32)
    x = x - jnp.mean(x, axis=1, keepdims=True)
    x = x * lax.rsqrt(jnp.sum(x * x, axis=1, keepdims=True) + 0.001)
    x = jnp.tanh(x).astype(jnp.bfloat16)
    x = jnp.dot(x, wt, preferred_element_type=jnp.float32)
    dxe2 = jnp.tanh(x)
    g2 = w * dxe2[:, :D]
    s2 = 0.5 * w * (dxe2[:, D:2 * D] + dxe2[:, 2 * D:3 * D]
                    + dxe2[:, 3 * D:4 * D] + dxe2[:, 4 * D:])
    md_ref[...] = g2 + s2
    ms_ref[...] = s2 - g2


def _edge_mlp(xe_attr, xs, xd, w1p, b, wt, *, interpret=False):
    nb = xe_attr.shape[0] // EDGE_BLOCK
    md, ms = pl.pallas_call(
        _edge_block_kernel,
        grid=(nb,),
        in_specs=[
            pl.BlockSpec((EDGE_BLOCK, ATTR), lambda i: (i, 0)),
            pl.BlockSpec((EDGE_BLOCK, D), lambda i: (i, 0)),
            pl.BlockSpec((EDGE_BLOCK, D), lambda i: (i, 0)),
            pl.BlockSpec((ATTR, D), lambda i: (0, 0)),
            pl.BlockSpec((1, D), lambda i: (0, 0)),
            pl.BlockSpec((D5, D5), lambda i: (0, 0)),
        ],
        out_specs=[
            pl.BlockSpec((EDGE_BLOCK, D), lambda i: (i, 0)),
            pl.BlockSpec((EDGE_BLOCK, D), lambda i: (i, 0)),
        ],
        out_shape=[
            jax.ShapeDtypeStruct((xe_attr.shape[0], D), jnp.float32),
            jax.ShapeDtypeStruct((xe_attr.shape[0], D), jnp.float32),
        ],
        interpret=interpret,
    )(xe_attr, xs, xd, w1p, b, wt)
    return md, ms


def kernel(xn, xe_attr, xe_src, xe_dst, fc1_w, fc1_b, dl_w1, dl_w2):
    del dl_w2
    w1p = fc1_w.T
    b = fc1_b.reshape(1, D)
    wt = dl_w1.T.astype(jnp.bfloat16)

    partials = []
    tok = jnp.zeros((8, D), jnp.float32)
    for s in range(N_SLAB):
        sl = slice(s * E_SLAB, (s + 1) * E_SLAB)
        src_s = xe_src[sl].reshape(NW, NCHUNK, CH)
        dst_s = xe_dst[sl].reshape(NW, NCHUNK, CH)
        xs, xd = _sc_gather(xn, src_s, dst_s)
        md, ms = _edge_mlp(xe_attr[sl], xs, xd, w1p, b, wt)
        p = _sc_scatter(md, ms, dst_s, src_s, tok)
        tok = p[0, :8]
        partials.append(p)
    acc = sum(p[0] + p[1] for p in partials)
    return acc[:N_NODES]


# speedup vs baseline: 1.5875x; 1.0542x over previous
import functools

import jax
import jax.numpy as jnp
from jax import lax
from jax.experimental import pallas as pl
from jax.experimental.pallas import tpu as pltpu
from jax.experimental.pallas import tpu_sc as plsc

N_NODES = 10000
N_EDGES = 320000
D = 128
D5 = 5 * D
ATTR = 33

EDGE_BLOCK = 2560

NC = 2
NS = 16
NW = NC * NS
CH = 80
NP = 10240

N_SLAB = 5
E_SLAB = N_EDGES // N_SLAB
EPW = E_SLAB // NW
NCHUNK = EPW // CH
GPR = 5
RB = GPR * CH
NGRP = EPW // RB
RB_S = CH
NGRP_S = EPW // RB_S

_SC_MESH = dict(core_axis_name="c", subcore_axis_name="s")


def _gather_body(xn_hbm, src_hbm, dst_hbm, xs_hbm, xd_hbm,
                 idxs_v, idxd_v, rows0_v, rows1_v,
                 gsem0, gsem1, wsem0, wsem1):
    cid = lax.axis_index("c")
    sid = lax.axis_index("s")
    wid = sid * NC + cid
    base = wid * EPW
    pltpu.sync_copy(src_hbm.at[wid], idxs_v)
    pltpu.sync_copy(dst_hbm.at[wid], idxd_v)

    rows = (rows0_v, rows1_v)
    gsems = (gsem0, gsem1)
    wsems = (wsem0, wsem1)
    nstep = 2 * NGRP
    plan = [(idxs_v, xs_hbm, t) if t < NGRP else (idxd_v, xd_hbm, t - NGRP)
            for t in range(nstep)]

    def fire_gathers(t):
        idxv, _, k = plan[t]
        buf, sem = rows[t % 2], gsems[t % 2]
        return [pltpu.async_copy(xn_hbm.at[idxv.at[GPR * k + b]],
                                 buf.at[pl.ds(b * CH, CH)], sem)
                for b in range(GPR)]

    wb = [None, None]
    pend = fire_gathers(0)
    for t in range(nstep):
        bt = t % 2
        for c in pend:
            c.wait()
        if t + 1 < nstep:
            if wb[1 - bt] is not None:
                wb[1 - bt].wait()
                wb[1 - bt] = None
            pend = fire_gathers(t + 1)
        _, out_hbm, k = plan[t]
        wb[bt] = pltpu.async_copy(
            rows[bt], out_hbm.at[pl.ds(base + k * RB, RB)], wsems[bt])
    for c in wb:
        if c is not None:
            c.wait()


def _sc_gather(xn, xe_src, xe_dst):
    return pl.kernel(
        _gather_body,
        out_type=[
            jax.ShapeDtypeStruct((E_SLAB, D), jnp.float32),
            jax.ShapeDtypeStruct((E_SLAB, D), jnp.float32),
        ],
        mesh=plsc.VectorSubcoreMesh(**_SC_MESH),
        scratch_types=[
            pltpu.VMEM((NCHUNK, CH), jnp.int32),
            pltpu.VMEM((NCHUNK, CH), jnp.int32),
            pltpu.VMEM((RB, D), jnp.float32),
            pltpu.VMEM((RB, D), jnp.float32),
            pltpu.SemaphoreType.DMA,
            pltpu.SemaphoreType.DMA,
            pltpu.SemaphoreType.DMA,
            pltpu.SemaphoreType.DMA,
        ],
    )(xn, xe_src, xe_dst)


def _scatter_body(md_hbm, ms_hbm, dst_hbm, src_hbm, tok_hbm, out_hbm,
                  idxd_v, idxs_v, rows0_v, rows1_v, acc_sh,
                  lsem0, lsem1, asem0, asem1):
    cid = lax.axis_index("c")
    sid = lax.axis_index("s")
    wid = sid * NC + cid
    base = wid * EPW
    zr = NP // NS
    pltpu.sync_copy(dst_hbm.at[wid], idxd_v)
    pltpu.sync_copy(src_hbm.at[wid], idxs_v)
    zvec = jnp.zeros((16,), jnp.float32)
    for r in range(16):
        for c in range(D // 16):
            rows0_v[r, pl.ds(c * 16, 16)] = zvec
    for z in range(zr // 16):
        pltpu.sync_copy(rows0_v.at[pl.ds(0, 16)],
                        acc_sh.at[pl.ds(sid * zr + z * 16, 16)])
    plsc.subcore_barrier()

    rows = (rows0_v, rows1_v)
    lsems = (lsem0, lsem1)
    asems = (asem0, asem1)
    nstep = 2 * NGRP_S
    plan = [(idxd_v, md_hbm, t) if t < NGRP_S else (idxs_v, ms_hbm, t - NGRP_S)
            for t in range(nstep)]

    def fire_load(t):
        _, m_hbm, k = plan[t]
        return pltpu.async_copy(m_hbm.at[pl.ds(base + k * RB_S, RB_S)],
                                rows[t % 2], lsems[t % 2])

    pend = fire_load(0)
    for t in range(nstep):
        bt = t % 2
        nxt = fire_load(t + 1) if t + 1 < nstep else None
        pend.wait()
        pend = nxt
        idxv, _, k = plan[t]
        pltpu.sync_copy(rows[bt], acc_sh.at[idxv.at[k]], add=True)
    plsc.subcore_barrier()
    pltpu.sync_copy(acc_sh.at[pl.ds(sid * zr, zr)],
                    out_hbm.at[cid].at[pl.ds(sid * zr, zr)])


def _sc_scatter(md, ms, xe_dst, xe_src, tok):
    return pl.kernel(
        _scatter_body,
        out_type=jax.ShapeDtypeStruct((NC, NP, D), jnp.float32),
        mesh=plsc.VectorSubcoreMesh(**_SC_MESH),
        scratch_types=[
            pltpu.VMEM((NCHUNK, CH), jnp.int32),
            pltpu.VMEM((NCHUNK, CH), jnp.int32),
            pltpu.VMEM((RB_S, D), jnp.float32),
            pltpu.VMEM((RB_S, D), jnp.float32),
            pltpu.VMEM_SHARED((NP, D), jnp.float32),
            pltpu.SemaphoreType.DMA,
            pltpu.SemaphoreType.DMA,
            pltpu.SemaphoreType.DMA,
            pltpu.SemaphoreType.DMA,
        ],
    )(md, ms, xe_dst, xe_src, tok)


def _edge_block_kernel(attr_ref, xs_ref, xd_ref, w1p_ref, b_ref, wt_ref,
                       md_ref, ms_ref):
    attr = attr_ref[...]
    w = jax.nn.silu(
        jnp.dot(attr, w1p_ref[...], preferred_element_type=jnp.float32)
        + b_ref[...])
    xs = xs_ref[...]
    xd = xd_ref[...]
    g = w * (xs - xd)
    a = 0.5 * w * (xs + xd)
    dxe = jnp.concatenate([g, a, g * a, g * g, a * a], axis=1)
    wt = wt_ref[...]
    x = jnp.tanh(dxe).astype(jnp.bfloat16)
    x = jnp.dot(x, wt, preferred_element_type=jnp.float32)
    x = x - jnp.mean(x, axis=1, keepdims=True)
    x = x * lax.rsqrt(jnp.sum(x * x, axis=1, keepdims=True) + 0.001)
    x = jnp.tanh(x).astype(jnp.bfloat16)
    x = jnp.dot(x, wt, preferred_element_type=jnp.float32)
    dxe2 = jnp.tanh(x)
    g2 = w * dxe2[:, :D]
    s2 = 0.5 * w * (dxe2[:, D:2 * D] + dxe2[:, 2 * D:3 * D]
                    + dxe2[:, 3 * D:4 * D] + dxe2[:, 4 * D:])
    md_ref[...] = g2 + s2
    ms_ref[...] = s2 - g2


def _edge_mlp(xe_attr, xs, xd, w1p, b, wt, *, interpret=False):
    nb = xe_attr.shape[0] // EDGE_BLOCK
    md, ms = pl.pallas_call(
        _edge_block_kernel,
        grid=(nb,),
        in_specs=[
            pl.BlockSpec((EDGE_BLOCK, ATTR), lambda i: (i, 0)),
            pl.BlockSpec((EDGE_BLOCK, D), lambda i: (i, 0)),
            pl.BlockSpec((EDGE_BLOCK, D), lambda i: (i, 0)),
            pl.BlockSpec((ATTR, D), lambda i: (0, 0)),
            pl.BlockSpec((1, D), lambda i: (0, 0)),
            pl.BlockSpec((D5, D5), lambda i: (0, 0)),
        ],
        out_specs=[
            pl.BlockSpec((EDGE_BLOCK, D), lambda i: (i, 0)),
            pl.BlockSpec((EDGE_BLOCK, D), lambda i: (i, 0)),
        ],
        out_shape=[
            jax.ShapeDtypeStruct((xe_attr.shape[0], D), jnp.float32),
            jax.ShapeDtypeStruct((xe_attr.shape[0], D), jnp.float32),
        ],
        interpret=interpret,
    )(xe_attr, xs, xd, w1p, b, wt)
    return md, ms


def kernel(xn, xe_attr, xe_src, xe_dst, fc1_w, fc1_b, dl_w1, dl_w2):
    del dl_w2
    w1p = fc1_w.T
    b = fc1_b.reshape(1, D)
    wt = dl_w1.T.astype(jnp.bfloat16)

    partials = []
    tok = jnp.zeros((8, D), jnp.float32)
    for s in range(N_SLAB):
        sl = slice(s * E_SLAB, (s + 1) * E_SLAB)
        src_s = xe_src[sl].reshape(NW, NCHUNK, CH)
        dst_s = xe_dst[sl].reshape(NW, NCHUNK, CH)
        xs, xd = _sc_gather(xn, src_s, dst_s)
        md, ms = _edge_mlp(xe_attr[sl], xs, xd, w1p, b, wt)
        p = _sc_scatter(md, ms, dst_s, src_s, tok)
        tok = p[0, :8]
        partials.append(p)
    acc = sum(p[0] + p[1] for p in partials)
    return acc[:N_NODES]


# speedup vs baseline: 1.6347x; 1.0297x over previous
import functools

import jax
import jax.numpy as jnp
from jax import lax
from jax.experimental import pallas as pl
from jax.experimental.pallas import tpu as pltpu
from jax.experimental.pallas import tpu_sc as plsc

N_NODES = 10000
N_EDGES = 320000
D = 128
D5 = 5 * D
ATTR = 33

EDGE_BLOCK = 2560

NC = 2
NS = 16
NW = NC * NS
CH = 80
NP = 10240

N_SLAB = 5
E_SLAB = N_EDGES // N_SLAB
EPW = E_SLAB // NW
NCHUNK = EPW // CH
GPR = 5
RB = GPR * CH
NGRP = EPW // RB
RB_S = CH
NGRP_S = EPW // RB_S

_SC_MESH = dict(core_axis_name="c", subcore_axis_name="s")


def _gather_body(slab, xn_hbm, src_hbm, dst_hbm, xs_hbm, xd_hbm,
                 idxs_v, idxd_v, rows0_v, rows1_v,
                 gsem0, gsem1, wsem0, wsem1):
    cid = lax.axis_index("c")
    sid = lax.axis_index("s")
    wid = sid * NC + cid
    base = wid * EPW
    pltpu.sync_copy(src_hbm.at[slab * NW + wid], idxs_v)
    pltpu.sync_copy(dst_hbm.at[slab * NW + wid], idxd_v)

    rows = (rows0_v, rows1_v)
    gsems = (gsem0, gsem1)
    wsems = (wsem0, wsem1)
    nstep = 2 * NGRP
    plan = [(idxs_v, xs_hbm, t) if t < NGRP else (idxd_v, xd_hbm, t - NGRP)
            for t in range(nstep)]

    def fire_gathers(t):
        idxv, _, k = plan[t]
        buf, sem = rows[t % 2], gsems[t % 2]
        return [pltpu.async_copy(xn_hbm.at[idxv.at[GPR * k + b]],
                                 buf.at[pl.ds(b * CH, CH)], sem)
                for b in range(GPR)]

    wb = [None, None]
    pend = fire_gathers(0)
    for t in range(nstep):
        bt = t % 2
        for c in pend:
            c.wait()
        if t + 1 < nstep:
            if wb[1 - bt] is not None:
                wb[1 - bt].wait()
                wb[1 - bt] = None
            pend = fire_gathers(t + 1)
        _, out_hbm, k = plan[t]
        wb[bt] = pltpu.async_copy(
            rows[bt], out_hbm.at[pl.ds(base + k * RB, RB)], wsems[bt])
    for c in wb:
        if c is not None:
            c.wait()


def _sc_gather(xn, xe_src, xe_dst, slab):
    return pl.kernel(
        functools.partial(_gather_body, slab),
        out_type=[
            jax.ShapeDtypeStruct((E_SLAB, D), jnp.float32),
            jax.ShapeDtypeStruct((E_SLAB, D), jnp.float32),
        ],
        mesh=plsc.VectorSubcoreMesh(**_SC_MESH),
        scratch_types=[
            pltpu.VMEM((NCHUNK, CH), jnp.int32),
            pltpu.VMEM((NCHUNK, CH), jnp.int32),
            pltpu.VMEM((RB, D), jnp.float32),
            pltpu.VMEM((RB, D), jnp.float32),
            pltpu.SemaphoreType.DMA,
            pltpu.SemaphoreType.DMA,
            pltpu.SemaphoreType.DMA,
            pltpu.SemaphoreType.DMA,
        ],
    )(xn, xe_src, xe_dst)


def _scatter_body(slab, md_hbm, ms_hbm, dst_hbm, src_hbm, tok_hbm, out_hbm,
                  idxd_v, idxs_v, rows0_v, rows1_v, acc_sh,
                  lsem0, lsem1, asem0, asem1):
    cid = lax.axis_index("c")
    sid = lax.axis_index("s")
    wid = sid * NC + cid
    base = wid * EPW
    zr = NP // NS
    pltpu.sync_copy(dst_hbm.at[slab * NW + wid], idxd_v)
    pltpu.sync_copy(src_hbm.at[slab * NW + wid], idxs_v)
    zvec = jnp.zeros((16,), jnp.float32)
    for r in range(16):
        for c in range(D // 16):
            rows0_v[r, pl.ds(c * 16, 16)] = zvec
    for z in range(zr // 16):
        pltpu.sync_copy(rows0_v.at[pl.ds(0, 16)],
                        acc_sh.at[pl.ds(sid * zr + z * 16, 16)])
    plsc.subcore_barrier()

    rows = (rows0_v, rows1_v)
    lsems = (lsem0, lsem1)
    asems = (asem0, asem1)
    nstep = 2 * NGRP_S
    plan = [(idxd_v, md_hbm, t) if t < NGRP_S else (idxs_v, ms_hbm, t - NGRP_S)
            for t in range(nstep)]

    def fire_load(t):
        _, m_hbm, k = plan[t]
        return pltpu.async_copy(m_hbm.at[pl.ds(base + k * RB_S, RB_S)],
                                rows[t % 2], lsems[t % 2])

    pend = fire_load(0)
    for t in range(nstep):
        bt = t % 2
        nxt = fire_load(t + 1) if t + 1 < nstep else None
        pend.wait()
        pend = nxt
        idxv, _, k = plan[t]
        pltpu.sync_copy(rows[bt], acc_sh.at[idxv.at[k]], add=True)
    plsc.subcore_barrier()
    pltpu.sync_copy(acc_sh.at[pl.ds(sid * zr, zr)],
                    out_hbm.at[cid].at[pl.ds(sid * zr, zr)])


def _sc_scatter(md, ms, xe_dst, xe_src, tok, slab):
    return pl.kernel(
        functools.partial(_scatter_body, slab),
        out_type=jax.ShapeDtypeStruct((NC, NP, D), jnp.float32),
        mesh=plsc.VectorSubcoreMesh(**_SC_MESH),
        scratch_types=[
            pltpu.VMEM((NCHUNK, CH), jnp.int32),
            pltpu.VMEM((NCHUNK, CH), jnp.int32),
            pltpu.VMEM((RB_S, D), jnp.float32),
            pltpu.VMEM((RB_S, D), jnp.float32),
            pltpu.VMEM_SHARED((NP, D), jnp.float32),
            pltpu.SemaphoreType.DMA,
            pltpu.SemaphoreType.DMA,
            pltpu.SemaphoreType.DMA,
            pltpu.SemaphoreType.DMA,
        ],
    )(md, ms, xe_dst, xe_src, tok)


def _edge_block_kernel(attr_ref, xs_ref, xd_ref, w1p_ref, b_ref, wt_ref,
                       md_ref, ms_ref):
    attr = attr_ref[...]
    w = jax.nn.silu(
        jnp.dot(attr, w1p_ref[...], preferred_element_type=jnp.float32)
        + b_ref[...])
    xs = xs_ref[...]
    xd = xd_ref[...]
    g = w * (xs - xd)
    a = 0.5 * w * (xs + xd)
    dxe = jnp.concatenate([g, a, g * a, g * g, a * a], axis=1)
    wt = wt_ref[...]
    x = jnp.tanh(dxe).astype(jnp.bfloat16)
    x = jnp.dot(x, wt, preferred_element_type=jnp.float32)
    x = x - jnp.mean(x, axis=1, keepdims=True)
    x = x * lax.rsqrt(jnp.sum(x * x, axis=1, keepdims=True) + 0.001)
    x = jnp.tanh(x).astype(jnp.bfloat16)
    x = jnp.dot(x, wt, preferred_element_type=jnp.float32)
    dxe2 = jnp.tanh(x)
    g2 = w * dxe2[:, :D]
    s2 = 0.5 * w * (dxe2[:, D:2 * D] + dxe2[:, 2 * D:3 * D]
                    + dxe2[:, 3 * D:4 * D] + dxe2[:, 4 * D:])
    md_ref[...] = g2 + s2
    ms_ref[...] = s2 - g2


def _edge_mlp(xe_attr, xs, xd, w1p, b, wt, slab=0, *, interpret=False):
    nb = E_SLAB // EDGE_BLOCK
    off = slab * nb
    md, ms = pl.pallas_call(
        _edge_block_kernel,
        grid=(nb,),
        in_specs=[
            pl.BlockSpec((EDGE_BLOCK, ATTR), lambda i: (i + off, 0)),
            pl.BlockSpec((EDGE_BLOCK, D), lambda i: (i, 0)),
            pl.BlockSpec((EDGE_BLOCK, D), lambda i: (i, 0)),
            pl.BlockSpec((ATTR, D), lambda i: (0, 0)),
            pl.BlockSpec((1, D), lambda i: (0, 0)),
            pl.BlockSpec((D5, D5), lambda i: (0, 0)),
        ],
        out_specs=[
            pl.BlockSpec((EDGE_BLOCK, D), lambda i: (i, 0)),
            pl.BlockSpec((EDGE_BLOCK, D), lambda i: (i, 0)),
        ],
        out_shape=[
            jax.ShapeDtypeStruct((E_SLAB, D), jnp.float32),
            jax.ShapeDtypeStruct((E_SLAB, D), jnp.float32),
        ],
        interpret=interpret,
    )(xe_attr, xs, xd, w1p, b, wt)
    return md, ms


def kernel(xn, xe_attr, xe_src, xe_dst, fc1_w, fc1_b, dl_w1, dl_w2):
    del dl_w2
    w1p = fc1_w.T
    b = fc1_b.reshape(1, D)
    wt = dl_w1.T.astype(jnp.bfloat16)

    src3 = xe_src.reshape(N_SLAB * NW, NCHUNK, CH)
    dst3 = xe_dst.reshape(N_SLAB * NW, NCHUNK, CH)
    partials = []
    tok = jnp.zeros((8, D), jnp.float32)
    for s in range(N_SLAB):
        xs, xd = _sc_gather(xn, src3, dst3, s)
        md, ms = _edge_mlp(xe_attr, xs, xd, w1p, b, wt, s)
        p = _sc_scatter(md, ms, dst3, src3, tok, s)
        tok = p[0, :8]
        partials.append(p)
    acc = sum(p[0] + p[1] for p in partials)
    return acc[:N_NODES]


# speedup vs baseline: 1.6421x; 1.0045x over previous
import functools

import jax
import jax.numpy as jnp
from jax import lax
from jax.experimental import pallas as pl
from jax.experimental.pallas import tpu as pltpu
from jax.experimental.pallas import tpu_sc as plsc

N_NODES = 10000
N_EDGES = 320000
D = 128
D5 = 5 * D
ATTR = 33

EDGE_BLOCK = 2560

NC = 2
NS = 16
NW = NC * NS
CH = 80
NP = 10240

N_SLAB = 5
E_SLAB = N_EDGES // N_SLAB
EPW = E_SLAB // NW
NCHUNK = EPW // CH
GPR = 5
RB = GPR * CH
NGRP = EPW // RB
RB_S = CH
NGRP_S = EPW // RB_S

_SC_MESH = dict(core_axis_name="c", subcore_axis_name="s")


def _gather_body(slab, xn_hbm, src_hbm, dst_hbm, xs_hbm, xd_hbm,
                 idxs_v, idxd_v, rows0_v, rows1_v,
                 gsem0, gsem1, wsem0, wsem1):
    cid = lax.axis_index("c")
    sid = lax.axis_index("s")
    wid = sid * NC + cid
    base = wid * EPW
    pltpu.sync_copy(src_hbm.at[slab * NW + wid], idxs_v)
    pltpu.sync_copy(dst_hbm.at[slab * NW + wid], idxd_v)

    rows = (rows0_v, rows1_v)
    gsems = (gsem0, gsem1)
    wsems = (wsem0, wsem1)
    nstep = 2 * NGRP
    plan = [(idxs_v, xs_hbm, t) if t < NGRP else (idxd_v, xd_hbm, t - NGRP)
            for t in range(nstep)]

    def fire_gathers(t):
        idxv, _, k = plan[t]
        buf, sem = rows[t % 2], gsems[t % 2]
        return [pltpu.async_copy(xn_hbm.at[idxv.at[GPR * k + b]],
                                 buf.at[pl.ds(b * CH, CH)], sem)
                for b in range(GPR)]

    wb = [None, None]
    pend = fire_gathers(0)
    for t in range(nstep):
        bt = t % 2
        for c in pend:
            c.wait()
        if t + 1 < nstep:
            if wb[1 - bt] is not None:
                wb[1 - bt].wait()
                wb[1 - bt] = None
            pend = fire_gathers(t + 1)
        _, out_hbm, k = plan[t]
        wb[bt] = pltpu.async_copy(
            rows[bt], out_hbm.at[pl.ds(base + k * RB, RB)], wsems[bt])
    for c in wb:
        if c is not None:
            c.wait()


def _sc_gather(xn, xe_src, xe_dst, slab):
    return pl.kernel(
        functools.partial(_gather_body, slab),
        out_type=[
            jax.ShapeDtypeStruct((E_SLAB, D), jnp.float32),
            jax.ShapeDtypeStruct((E_SLAB, D), jnp.float32),
        ],
        mesh=plsc.VectorSubcoreMesh(**_SC_MESH),
        scratch_types=[
            pltpu.VMEM((NCHUNK, CH), jnp.int32),
            pltpu.VMEM((NCHUNK, CH), jnp.int32),
            pltpu.VMEM((RB, D), jnp.float32),
            pltpu.VMEM((RB, D), jnp.float32),
            pltpu.SemaphoreType.DMA,
            pltpu.SemaphoreType.DMA,
            pltpu.SemaphoreType.DMA,
            pltpu.SemaphoreType.DMA,
        ],
    )(xn, xe_src, xe_dst)


def _scatter_body(slab, md_hbm, ms_hbm, dst_hbm, src_hbm, out_hbm,
                  idxd_v, idxs_v, rows0_v, rows1_v, acc_sh,
                  lsem0, lsem1, asem0, asem1):
    cid = lax.axis_index("c")
    sid = lax.axis_index("s")
    wid = sid * NC + cid
    base = wid * EPW
    zr = NP // NS
    pltpu.sync_copy(dst_hbm.at[slab * NW + wid], idxd_v)
    pltpu.sync_copy(src_hbm.at[slab * NW + wid], idxs_v)
    zvec = jnp.zeros((16,), jnp.float32)
    for r in range(16):
        for c in range(D // 16):
            rows0_v[r, pl.ds(c * 16, 16)] = zvec
    for z in range(zr // 16):
        pltpu.sync_copy(rows0_v.at[pl.ds(0, 16)],
                        acc_sh.at[pl.ds(sid * zr + z * 16, 16)])
    plsc.subcore_barrier()

    rows = (rows0_v, rows1_v)
    lsems = (lsem0, lsem1)
    asems = (asem0, asem1)
    nstep = 2 * NGRP_S
    plan = [(idxd_v, md_hbm, t) if t < NGRP_S else (idxs_v, ms_hbm, t - NGRP_S)
            for t in range(nstep)]

    def fire_load(t):
        _, m_hbm, k = plan[t]
        return pltpu.async_copy(m_hbm.at[pl.ds(base + k * RB_S, RB_S)],
                                rows[t % 2], lsems[t % 2])

    pend = fire_load(0)
    for t in range(nstep):
        bt = t % 2
        nxt = fire_load(t + 1) if t + 1 < nstep else None
        pend.wait()
        pend = nxt
        idxv, _, k = plan[t]
        pltpu.sync_copy(rows[bt], acc_sh.at[idxv.at[k]], add=True)
    plsc.subcore_barrier()
    pltpu.sync_copy(acc_sh.at[pl.ds(sid * zr, zr)],
                    out_hbm.at[cid].at[pl.ds(sid * zr, zr)])


def _sc_scatter(md, ms, xe_dst, xe_src, slab):
    return pl.kernel(
        functools.partial(_scatter_body, slab),
        out_type=jax.ShapeDtypeStruct((NC, NP, D), jnp.float32),
        mesh=plsc.VectorSubcoreMesh(**_SC_MESH),
        scratch_types=[
            pltpu.VMEM((NCHUNK, CH), jnp.int32),
            pltpu.VMEM((NCHUNK, CH), jnp.int32),
            pltpu.VMEM((RB_S, D), jnp.float32),
            pltpu.VMEM((RB_S, D), jnp.float32),
            pltpu.VMEM_SHARED((NP, D), jnp.float32),
            pltpu.SemaphoreType.DMA,
            pltpu.SemaphoreType.DMA,
            pltpu.SemaphoreType.DMA,
            pltpu.SemaphoreType.DMA,
        ],
    )(md, ms, xe_dst, xe_src)


def _edge_block_kernel(attr_ref, xs_ref, xd_ref, w1p_ref, b_ref, wt_ref,
                       md_ref, ms_ref):
    attr = attr_ref[...]
    w = jax.nn.silu(
        jnp.dot(attr, w1p_ref[...], preferred_element_type=jnp.float32)
        + b_ref[...])
    xs = xs_ref[...]
    xd = xd_ref[...]
    g = w * (xs - xd)
    a = 0.5 * w * (xs + xd)
    dxe = jnp.concatenate([g, a, g * a, g * g, a * a], axis=1)
    wt = wt_ref[...]
    x = jnp.tanh(dxe).astype(jnp.bfloat16)
    x = jnp.dot(x, wt, preferred_element_type=jnp.float32)
    x = x - jnp.mean(x, axis=1, keepdims=True)
    x = x * lax.rsqrt(jnp.sum(x * x, axis=1, keepdims=True) + 0.001)
    x = jnp.tanh(x).astype(jnp.bfloat16)
    x = jnp.dot(x, wt, preferred_element_type=jnp.float32)
    dxe2 = jnp.tanh(x)
    g2 = w * dxe2[:, :D]
    s2 = 0.5 * w * (dxe2[:, D:2 * D] + dxe2[:, 2 * D:3 * D]
                    + dxe2[:, 3 * D:4 * D] + dxe2[:, 4 * D:])
    md_ref[...] = g2 + s2
    ms_ref[...] = s2 - g2


def _edge_mlp(xe_attr, xs, xd, w1p, b, wt, slab=0, *, interpret=False):
    nb = E_SLAB // EDGE_BLOCK
    off = slab * nb
    md, ms = pl.pallas_call(
        _edge_block_kernel,
        grid=(nb,),
        in_specs=[
            pl.BlockSpec((EDGE_BLOCK, ATTR), lambda i: (i + off, 0)),
            pl.BlockSpec((EDGE_BLOCK, D), lambda i: (i, 0)),
            pl.BlockSpec((EDGE_BLOCK, D), lambda i: (i, 0)),
            pl.BlockSpec((ATTR, D), lambda i: (0, 0)),
            pl.BlockSpec((1, D), lambda i: (0, 0)),
            pl.BlockSpec((D5, D5), lambda i: (0, 0)),
        ],
        out_specs=[
            pl.BlockSpec((EDGE_BLOCK, D), lambda i: (i, 0)),
            pl.BlockSpec((EDGE_BLOCK, D), lambda i: (i, 0)),
        ],
        out_shape=[
            jax.ShapeDtypeStruct((E_SLAB, D), jnp.float32),
            jax.ShapeDtypeStruct((E_SLAB, D), jnp.float32),
        ],
        interpret=interpret,
    )(xe_attr, xs, xd, w1p, b, wt)
    return md, ms


def kernel(xn, xe_attr, xe_src, xe_dst, fc1_w, fc1_b, dl_w1, dl_w2):
    del dl_w2
    w1p = fc1_w.T
    b = fc1_b.reshape(1, D)
    wt = dl_w1.T.astype(jnp.bfloat16)

    src3 = xe_src.reshape(N_SLAB * NW, NCHUNK, CH)
    dst3 = xe_dst.reshape(N_SLAB * NW, NCHUNK, CH)
    partials = []
    for s in range(N_SLAB):
        xs, xd = _sc_gather(xn, src3, dst3, s)
        md, ms = _edge_mlp(xe_attr, xs, xd, w1p, b, wt, s)
        if partials:
            md, _ = lax.optimization_barrier((md, partials[-1]))
        partials.append(_sc_scatter(md, ms, dst3, src3, s))
    acc = sum(p[0] + p[1] for p in partials)
    return acc[:N_NODES]
